# baseline (speedup 1.0000x reference)
import jax
import jax.numpy as jnp
from jax.experimental import pallas as pl

_MEDNUM = 5000
_D = 128
_BM = 200
_BK = 2048
_K = 2 * _MEDNUM
_KPAD = 10240


def _gcn_block(adj_ref, emb_ref, out_ref):
    k = pl.program_id(1)
    a = adj_ref[...]
    cols = k * _BK + jax.lax.broadcasted_iota(jnp.int32, (_BM, _BK), 1)
    a = jnp.where(cols < _K, a, 0.0)
    e = emb_ref[pl.ds(k * _BK, _BK), :]
    h = jnp.dot(a, e, preferred_element_type=jnp.float32)

    @pl.when(k == 0)
    def _():
        out_ref[...] = h

    @pl.when(k > 0)
    def _():
        out_ref[...] += h

    @pl.when(k == pl.num_programs(1) - 1)
    def _():
        out_ref[...] = 4.0 * jnp.maximum(out_ref[...], 0.0)


def kernel(adj1, m1Embed, m2Embed):
    embeds = jnp.concatenate([m1Embed, m2Embed], axis=0)
    embeds = jnp.pad(embeds, ((0, _KPAD - _K), (0, 0)))
    return pl.pallas_call(
        _gcn_block,
        grid=(pl.cdiv(_MEDNUM, _BM), _KPAD // _BK),
        in_specs=[
            pl.BlockSpec((_BM, _BK), lambda i, j: (i, j)),
            pl.BlockSpec((_KPAD, _D), lambda i, j: (0, 0)),
        ],
        out_specs=pl.BlockSpec((_BM, _D), lambda i, j: (i, 0)),
        out_shape=jax.ShapeDtypeStruct((_MEDNUM, _D), jnp.float32),
    )(adj1, embeds)
